# initial kernel scaffold (unmeasured)
import jax
import jax.numpy as jnp
from jax import lax
from jax.experimental import pallas as pl
from jax.experimental.pallas import tpu as pltpu

M = 2048
N = 2048
K_SHARD = 8192
MX = M // 2
NT = 512


def kernel(dy, W):
    my_x = lax.axis_index("x")

    dy_bf = dy.astype(jnp.bfloat16)
    w_bf = W.astype(jnp.bfloat16)
    dy_rows = lax.dynamic_slice_in_dim(dy_bf, my_x * MX, MX, axis=0)

    def body(dy_ref, w_hbm, out_ref,
             p_send, p_recv, g_send, g_recv, w_tile,
             dma_sem, send_sems, recv_sems):
        my_x = lax.axis_index("x")
        my_y = lax.axis_index("y")
        y_nbr = (my_x, 1 - my_y)
        x_nbr = (1 - my_x, my_y)

        barrier = pltpu.get_barrier_semaphore()
        for nbr in (y_nbr, x_nbr):
            pl.semaphore_signal(barrier, inc=1, device_id=nbr,
                                device_id_type=pl.DeviceIdType.MESH)
        pl.semaphore_wait(barrier, 2)

        for t in range(N // NT):
            cp = pltpu.make_async_copy(
                w_hbm.at[pl.ds(t * NT, NT), :], w_tile, dma_sem)
            cp.start()
            cp.wait()
            acc = lax.dot_general(
                dy_ref[:, :], w_tile[:, :],
                (((1,), (1,)), ((), ())),
                preferred_element_type=jnp.float32)
            p_send[:, pl.ds(t * NT, NT)] = acc.astype(jnp.bfloat16)

        rdma_y = pltpu.make_async_remote_copy(
            src_ref=p_send, dst_ref=p_recv,
            send_sem=send_sems.at[0], recv_sem=recv_sems.at[0],
            device_id=y_nbr, device_id_type=pl.DeviceIdType.MESH)
        rdma_y.start()
        rdma_y.wait()

        red = p_send[:, :].astype(jnp.float32) + p_recv[:, :].astype(jnp.float32)
        out_ref[pl.ds(my_x * MX, MX), :] = red
        g_send[:, :] = red.astype(jnp.bfloat16)

        rdma_x = pltpu.make_async_remote_copy(
            src_ref=g_send, dst_ref=g_recv,
            send_sem=send_sems.at[1], recv_sem=recv_sems.at[1],
            device_id=x_nbr, device_id_type=pl.DeviceIdType.MESH)
        rdma_x.start()
        rdma_x.wait()

        other_x = 1 - my_x
        out_ref[pl.ds(other_x * MX, MX), :] = g_recv[:, :].astype(jnp.float32)

    return pl.pallas_call(
        body,
        out_shape=jax.ShapeDtypeStruct((M, N), jnp.float32),
        in_specs=[
            pl.BlockSpec(memory_space=pltpu.VMEM),
            pl.BlockSpec(memory_space=pltpu.ANY),
        ],
        out_specs=pl.BlockSpec(memory_space=pltpu.VMEM),
        scratch_shapes=[
            pltpu.VMEM((MX, N), jnp.bfloat16),
            pltpu.VMEM((MX, N), jnp.bfloat16),
            pltpu.VMEM((MX, N), jnp.bfloat16),
            pltpu.VMEM((MX, N), jnp.bfloat16),
            pltpu.VMEM((NT, K_SHARD), jnp.bfloat16),
            pltpu.SemaphoreType.DMA,
            pltpu.SemaphoreType.DMA((2,)),
            pltpu.SemaphoreType.DMA((2,)),
        ],
        compiler_params=pltpu.CompilerParams(collective_id=0),
    )(dy_rows, w_bf)


# baseline (device time: 250233 ns/iter reference)
import jax
import jax.numpy as jnp
from jax import lax
from jax.experimental import pallas as pl
from jax.experimental.pallas import tpu as pltpu

M = 2048
N = 2048
K_SHARD = 8192
MX = M // 2
NT = 256
NC = N // 2


def kernel(dy, W):
    my_x = lax.axis_index("x")

    dy_bf = dy.astype(jnp.bfloat16)
    w_bf = W.astype(jnp.bfloat16)
    dy_rows = lax.dynamic_slice_in_dim(dy_bf, my_x * MX, MX, axis=0)

    def body(dy_ref, w_hbm, out_ref,
             p_send, p_recv, g_recv, w_tile,
             dma_sem, send_sems, recv_sems):
        my_x = lax.axis_index("x")
        my_y = lax.axis_index("y")
        y_nbr = (my_x, 1 - my_y)
        x_nbr = (1 - my_x, my_y)

        barrier = pltpu.get_barrier_semaphore()
        for nbr in (y_nbr, x_nbr):
            pl.semaphore_signal(barrier, inc=1, device_id=nbr,
                                device_id_type=pl.DeviceIdType.MESH)
        pl.semaphore_wait(barrier, 2)

        for t in range(N // NT):
            cp = pltpu.make_async_copy(
                w_hbm.at[pl.ds(t * NT, NT), :], w_tile, dma_sem)
            cp.start()
            cp.wait()
            acc = lax.dot_general(
                dy_ref[:, :], w_tile[:, :],
                (((1,), (1,)), ((), ())),
                preferred_element_type=jnp.float32)
            p_send[:, pl.ds(t * NT, NT)] = acc.astype(jnp.bfloat16)

        rdma_y = pltpu.make_async_remote_copy(
            src_ref=p_send, dst_ref=p_recv,
            send_sem=send_sems.at[0], recv_sem=recv_sems.at[0],
            device_id=y_nbr, device_id_type=pl.DeviceIdType.MESH)
        rdma_y.start()
        rdma_y.wait()

        for c in range(N // NC):
            red = (p_send[:, pl.ds(c * NC, NC)].astype(jnp.float32)
                   + p_recv[:, pl.ds(c * NC, NC)].astype(jnp.float32))
            out_ref[pl.ds(my_x * MX, MX), pl.ds(c * NC, NC)] = red
            p_send[:, pl.ds(c * NC, NC)] = red.astype(jnp.bfloat16)

        rdma_x = pltpu.make_async_remote_copy(
            src_ref=p_send, dst_ref=g_recv,
            send_sem=send_sems.at[1], recv_sem=recv_sems.at[1],
            device_id=x_nbr, device_id_type=pl.DeviceIdType.MESH)
        rdma_x.start()
        rdma_x.wait()

        other_x = 1 - my_x
        for c in range(N // NC):
            out_ref[pl.ds(other_x * MX, MX), pl.ds(c * NC, NC)] = (
                g_recv[:, pl.ds(c * NC, NC)].astype(jnp.float32))

    return pl.pallas_call(
        body,
        out_shape=jax.ShapeDtypeStruct((M, N), jnp.float32),
        in_specs=[
            pl.BlockSpec(memory_space=pltpu.VMEM),
            pl.BlockSpec(memory_space=pl.ANY),
        ],
        out_specs=pl.BlockSpec(memory_space=pltpu.VMEM),
        scratch_shapes=[
            pltpu.VMEM((MX, N), jnp.bfloat16),
            pltpu.VMEM((MX, N), jnp.bfloat16),
            pltpu.VMEM((MX, N), jnp.bfloat16),
            pltpu.VMEM((NT, K_SHARD), jnp.bfloat16),
            pltpu.SemaphoreType.DMA,
            pltpu.SemaphoreType.DMA((2,)),
            pltpu.SemaphoreType.DMA((2,)),
        ],
        compiler_params=pltpu.CompilerParams(collective_id=0),
    )(dy_rows, w_bf)


# device time: 123484 ns/iter; 2.0264x vs baseline; 2.0264x over previous
import jax
import jax.numpy as jnp
from jax import lax
from jax.experimental import pallas as pl
from jax.experimental.pallas import tpu as pltpu

M = 2048
N = 2048
K_SHARD = 8192
MX = M // 2
NT = 256
T = N // NT
LAG = 2
QD = MX // NT


def kernel(dy, W):

    def body(dy_hbm, w_hbm, out_hbm,
             dy_bf, stage, p_send, p_recv, g_recv, ostage,
             stage_sems, out_sems,
             y_send, y_recv, x_send, x_recv):
        my_x = lax.axis_index("x")
        my_y = lax.axis_index("y")
        y_nbr = (my_x, 1 - my_y)
        x_nbr = (1 - my_x, my_y)
        my_row0 = my_x * MX
        other_row0 = (1 - my_x) * MX

        barrier = pltpu.get_barrier_semaphore()
        for nbr in (y_nbr, x_nbr):
            pl.semaphore_signal(barrier, inc=1, device_id=nbr,
                                device_id_type=pl.DeviceIdType.MESH)
        pl.semaphore_wait(barrier, 2)

        def stage_dma(src_2d, slot):
            return pltpu.make_async_copy(src_2d, stage.at[slot],
                                         stage_sems.at[slot])

        def dy_dma(q, slot):
            return stage_dma(
                dy_hbm.at[pl.ds(my_row0 + q * NT, NT), :], slot)

        def w_dma(t, slot):
            return stage_dma(w_hbm.at[pl.ds(t * NT, NT), :], slot)

        def y_desc(t):
            return pltpu.make_async_remote_copy(
                src_ref=p_send.at[t], dst_ref=p_recv.at[t],
                send_sem=y_send.at[t], recv_sem=y_recv.at[t],
                device_id=y_nbr, device_id_type=pl.DeviceIdType.MESH)

        def x_desc(t):
            return pltpu.make_async_remote_copy(
                src_ref=p_send.at[t], dst_ref=g_recv.at[t],
                send_sem=x_send.at[t], recv_sem=x_recv.at[t],
                device_id=x_nbr, device_id_type=pl.DeviceIdType.MESH)

        dy_dma(0, 0).start()
        dy_dma(1, 1).start()
        for q in range(QD):
            s = q % 2
            pltpu.make_async_copy(
                dy_hbm.at[pl.ds(my_row0 + q * NT, NT), :], stage.at[s],
                stage_sems.at[s]).wait()
            dy_bf[pl.ds(q * NT, NT), :] = stage[s].astype(jnp.bfloat16)
            nxt = q + 2
            if nxt < QD:
                dy_dma(nxt, s).start()
            elif nxt < QD + 2:
                w_dma(nxt - QD, s).start()

        ostage_busy = [False, False]

        def flush_out(val_f32, row0, t):
            s = t % 2
            if ostage_busy[s]:
                pltpu.make_async_copy(
                    ostage.at[s],
                    out_hbm.at[pl.ds(0, MX), pl.ds(0, NT)],
                    out_sems.at[s]).wait()
            ostage[s] = val_f32
            pltpu.make_async_copy(
                ostage.at[s],
                out_hbm.at[pl.ds(row0, MX), pl.ds(t * NT, NT)],
                out_sems.at[s]).start()
            ostage_busy[s] = True

        def process(u):
            y_desc(u).wait_recv()
            y_desc(u).wait_send()
            red = (p_send[u].astype(jnp.float32)
                   + p_recv[u].astype(jnp.float32))
            flush_out(red, my_row0, u)
            p_send[u] = red.astype(jnp.bfloat16)
            x_desc(u).start()

        for t in range(T):
            s = t % 2
            pltpu.make_async_copy(
                w_hbm.at[pl.ds(t * NT, NT), :], stage.at[s],
                stage_sems.at[s]).wait()
            acc = lax.dot_general(
                dy_bf[:, :], stage[s].astype(jnp.bfloat16),
                (((1,), (1,)), ((), ())),
                preferred_element_type=jnp.float32)
            p_send[t] = acc.astype(jnp.bfloat16)
            y_desc(t).start()
            if t + 2 < T:
                w_dma(t + 2, s).start()
            if t >= LAG:
                process(t - LAG)
        for u in range(T - LAG, T):
            process(u)

        for t in range(T):
            x_desc(t).wait_recv()
            flush_out(g_recv[t].astype(jnp.float32), other_row0, t)
        for t in range(T):
            x_desc(t).wait_send()
        for s in range(2):
            if ostage_busy[s]:
                pltpu.make_async_copy(
                    ostage.at[s],
                    out_hbm.at[pl.ds(0, MX), pl.ds(0, NT)],
                    out_sems.at[s]).wait()

    return pl.pallas_call(
        body,
        out_shape=jax.ShapeDtypeStruct((M, N), jnp.float32),
        in_specs=[
            pl.BlockSpec(memory_space=pl.ANY),
            pl.BlockSpec(memory_space=pl.ANY),
        ],
        out_specs=pl.BlockSpec(memory_space=pl.ANY),
        scratch_shapes=[
            pltpu.VMEM((MX, K_SHARD), jnp.bfloat16),
            pltpu.VMEM((2, NT, K_SHARD), jnp.float32),
            pltpu.VMEM((T, MX, NT), jnp.bfloat16),
            pltpu.VMEM((T, MX, NT), jnp.bfloat16),
            pltpu.VMEM((T, MX, NT), jnp.bfloat16),
            pltpu.VMEM((2, MX, NT), jnp.float32),
            pltpu.SemaphoreType.DMA((2,)),
            pltpu.SemaphoreType.DMA((2,)),
            pltpu.SemaphoreType.DMA((T,)),
            pltpu.SemaphoreType.DMA((T,)),
            pltpu.SemaphoreType.DMA((T,)),
            pltpu.SemaphoreType.DMA((T,)),
        ],
        compiler_params=pltpu.CompilerParams(
            collective_id=0,
            vmem_limit_bytes=100 * 1024 * 1024,
        ),
    )(dy, W)


# device time: 91975 ns/iter; 2.7207x vs baseline; 1.3426x over previous
import jax
import jax.numpy as jnp
from jax import lax
from jax.experimental import pallas as pl
from jax.experimental.pallas import tpu as pltpu

M = 2048
N = 2048
K_SHARD = 8192
MX = M // 2
MH = MX // 2
NT = 256
T = N // NT
LAG = 2
XLAG = 2
QD = MX // NT


def kernel(dy, W):

    def body(dy_hbm, w_hbm, out_hbm,
             dy_bf, stage, p_send, p_recv, g_recv, ostage,
             stage_sems, out_sems,
             y_send, y_recv, x_send, x_recv):
        my_x = lax.axis_index("x")
        my_y = lax.axis_index("y")
        y_nbr = (my_x, 1 - my_y)
        x_nbr = (1 - my_x, my_y)
        my_row0 = my_x * MX
        other_row0 = (1 - my_x) * MX

        def stage_dma(src_2d, slot):
            return pltpu.make_async_copy(src_2d, stage.at[slot],
                                         stage_sems.at[slot])

        def dy_dma(q, slot):
            return stage_dma(
                dy_hbm.at[pl.ds(my_row0 + q * NT, NT), :], slot)

        def w_dma(t, slot):
            return stage_dma(w_hbm.at[pl.ds(t * NT, NT), :], slot)

        def y_desc(t):
            return pltpu.make_async_remote_copy(
                src_ref=p_send.at[t], dst_ref=p_recv.at[t],
                send_sem=y_send.at[t], recv_sem=y_recv.at[t],
                device_id=y_nbr, device_id_type=pl.DeviceIdType.MESH)

        def x_desc(t):
            return pltpu.make_async_remote_copy(
                src_ref=p_send.at[t], dst_ref=g_recv.at[t],
                send_sem=x_send.at[t], recv_sem=x_recv.at[t],
                device_id=x_nbr, device_id_type=pl.DeviceIdType.MESH)

        dy_dma(0, 0).start()
        dy_dma(1, 1).start()

        barrier = pltpu.get_barrier_semaphore()
        for nbr in (y_nbr, x_nbr):
            pl.semaphore_signal(barrier, inc=1, device_id=nbr,
                                device_id_type=pl.DeviceIdType.MESH)
        pl.semaphore_wait(barrier, 2)

        for q in range(QD):
            s = q % 2
            dy_dma(q, s).wait()
            dy_bf[pl.ds(q * NT, NT), :] = stage[s].astype(jnp.bfloat16)
            nxt = q + 2
            if nxt < QD:
                dy_dma(nxt, s).start()
            elif nxt < QD + 2:
                w_dma(nxt - QD, s).start()

        ostage_busy = [False, False]

        def flush_out(val, row0, t):
            s = t % 2
            if ostage_busy[s]:
                pltpu.make_async_copy(
                    ostage.at[s],
                    out_hbm.at[pl.ds(0, MX), pl.ds(0, NT)],
                    out_sems.at[s]).wait()
            ostage[s] = val
            pltpu.make_async_copy(
                ostage.at[s],
                out_hbm.at[pl.ds(row0, MX), pl.ds(t * NT, NT)],
                out_sems.at[s]).start()
            ostage_busy[s] = True

        def drain_x(v):
            x_desc(v).wait_recv()
            flush_out(g_recv[v].astype(jnp.float32), other_row0, v)

        def process(u):
            y_desc(u).wait_recv()
            y_desc(u).wait_send()
            red = (p_send[u].astype(jnp.float32)
                   + p_recv[u].astype(jnp.float32))
            flush_out(red, my_row0, u)
            p_send[u] = red.astype(jnp.bfloat16)
            x_desc(u).start()
            if u >= XLAG:
                drain_x(u - XLAG)

        for t in range(T):
            s = t % 2
            w_dma(t, s).wait()
            w_bf = stage[s].astype(jnp.bfloat16)
            dd = (((1,), (1,)), ((), ()))
            acc_top = lax.dot_general(
                dy_bf[pl.ds(0, MH), :], w_bf, dd,
                preferred_element_type=jnp.float32)
            acc_bot = lax.dot_general(
                dy_bf[pl.ds(MH, MH), :], w_bf, dd,
                preferred_element_type=jnp.float32)
            p_send[t, pl.ds(0, MH)] = acc_top.astype(jnp.bfloat16)
            p_send[t, pl.ds(MH, MH)] = acc_bot.astype(jnp.bfloat16)
            y_desc(t).start()
            if t + 2 < T:
                w_dma(t + 2, s).start()
            if t >= LAG:
                process(t - LAG)
        for u in range(T - LAG, T):
            process(u)

        for v in range(T - XLAG, T):
            drain_x(v)
        for t in range(T):
            x_desc(t).wait_send()
        for s in range(2):
            if ostage_busy[s]:
                pltpu.make_async_copy(
                    ostage.at[s],
                    out_hbm.at[pl.ds(0, MX), pl.ds(0, NT)],
                    out_sems.at[s]).wait()

    return pl.pallas_call(
        body,
        out_shape=jax.ShapeDtypeStruct((M, N), jnp.float32),
        in_specs=[
            pl.BlockSpec(memory_space=pl.ANY),
            pl.BlockSpec(memory_space=pl.ANY),
        ],
        out_specs=pl.BlockSpec(memory_space=pl.ANY),
        scratch_shapes=[
            pltpu.VMEM((MX, K_SHARD), jnp.bfloat16),
            pltpu.VMEM((2, NT, K_SHARD), jnp.float32),
            pltpu.VMEM((T, MX, NT), jnp.bfloat16),
            pltpu.VMEM((T, MX, NT), jnp.bfloat16),
            pltpu.VMEM((T, MX, NT), jnp.bfloat16),
            pltpu.VMEM((2, MX, NT), jnp.float32),
            pltpu.SemaphoreType.DMA((2,)),
            pltpu.SemaphoreType.DMA((2,)),
            pltpu.SemaphoreType.DMA((T,)),
            pltpu.SemaphoreType.DMA((T,)),
            pltpu.SemaphoreType.DMA((T,)),
            pltpu.SemaphoreType.DMA((T,)),
        ],
        compiler_params=pltpu.CompilerParams(
            collective_id=0,
            vmem_limit_bytes=100 * 1024 * 1024,
        ),
    )(dy, W)
